# Initial kernel scaffold; baseline (speedup 1.0000x reference)
import functools

import jax
import jax.numpy as jnp
from jax import lax
from jax.experimental import pallas as pl
from jax.experimental.pallas import tpu as pltpu
from jax.experimental.pallas import tpu_sc as plsc

N = 10000
E = 320000
D = 128
NC, NS = 2, 16
NW = NC * NS
E_PER = E // NW
CHUNK = 80
NCHUNK = E_PER // CHUNK
ROWS_PER_TILE = N // NS

_MESH = plsc.VectorSubcoreMesh(core_axis_name="c", subcore_axis_name="s")


@functools.partial(
    pl.kernel,
    mesh=_MESH,
    out_type=jax.ShapeDtypeStruct((NC * N,), jnp.float32),
    scratch_types=[
        pltpu.VMEM((CHUNK,), jnp.int32),
        pltpu.VMEM((CHUNK,), jnp.float32),
        pltpu.VMEM_SHARED((N,), jnp.float32),
    ],
)
def _sc_degree(dst_hbm, zeros_hbm, deg_hbm, dst_v, ones_v, deg_sh):
    c = lax.axis_index("c")
    s = lax.axis_index("s")
    wid = s * NC + c
    for j in range(CHUNK // 16):
        ones_v[pl.ds(j * 16, 16)] = jnp.ones((16,), jnp.float32)
    @pl.when(s == 0)
    def _():
        pltpu.sync_copy(zeros_hbm, deg_sh)
    plsc.subcore_barrier()

    def step(i, carry):
        pltpu.sync_copy(dst_hbm.at[wid, i], dst_v)
        pltpu.sync_copy(ones_v, deg_sh.at[dst_v], add=True)
        return carry

    lax.fori_loop(0, NCHUNK, step, 0)
    plsc.subcore_barrier()

    @pl.when(s == 0)
    def _():
        pltpu.sync_copy(deg_sh, deg_hbm.at[pl.ds(c * N, N)])


@functools.partial(
    pl.kernel,
    mesh=_MESH,
    out_type=jax.ShapeDtypeStruct((NC * N, D), jnp.float32),
    scratch_types=[
        pltpu.VMEM((CHUNK,), jnp.int32),
        pltpu.VMEM((CHUNK,), jnp.int32),
        pltpu.VMEM((CHUNK, D), jnp.float32),
        pltpu.VMEM_SHARED((N, D), jnp.float32),
        pltpu.SemaphoreType.DMA,
    ],
)
def _sc_scatter(g_hbm, src_hbm, dst_hbm, zrows_hbm, acc_hbm,
                src_v, dst_v, rows_v, acc_sh, sem):
    c = lax.axis_index("c")
    s = lax.axis_index("s")
    wid = s * NC + c
    r0 = s * ROWS_PER_TILE
    pltpu.sync_copy(zrows_hbm, acc_sh.at[pl.ds(r0, ROWS_PER_TILE)])
    plsc.subcore_barrier()

    def step(i, carry):
        pltpu.sync_copy(src_hbm.at[wid, i], src_v)
        pltpu.sync_copy(dst_hbm.at[wid, i], dst_v)
        pltpu.async_copy(g_hbm.at[src_v], rows_v, sem).wait()
        pltpu.sync_copy(rows_v, acc_sh.at[dst_v], add=True)
        return carry

    lax.fori_loop(0, NCHUNK, step, 0)
    plsc.subcore_barrier()
    pltpu.sync_copy(acc_sh.at[pl.ds(r0, ROWS_PER_TILE)],
                    acc_hbm.at[pl.ds(c * N + r0, ROWS_PER_TILE)])


BLK = 1000


def _tc_k1_body(x_ref, w_ref, d0_ref, d1_ref, g_ref, dis_ref):
    h = jnp.dot(x_ref[...], w_ref[...], preferred_element_type=jnp.float32)
    dis = lax.rsqrt(d0_ref[...] + d1_ref[...] + 1.0)
    g_ref[...] = h * dis
    dis_ref[...] = dis


def _tc_k2_body(acc_ref, g1_ref, dis_ref, b1_ref, w2_ref, g2_ref):
    dis = dis_ref[...]
    out1 = dis * (acc_ref[0] + acc_ref[1] + g1_ref[...]) + b1_ref[...]
    g2_ref[...] = jnp.dot(out1, w2_ref[...],
                          preferred_element_type=jnp.float32) * dis


def _tc_k3_body(acc_ref, g2_ref, dis_ref, b2_ref, out_ref):
    out_ref[...] = (dis_ref[...] * (acc_ref[0] + acc_ref[1] + g2_ref[...])
                    + b2_ref[...])


_row_blk = pl.BlockSpec((BLK, D), lambda i: (i, 0))
_col_blk = pl.BlockSpec((BLK, 1), lambda i: (i, 0))
_mat_blk = pl.BlockSpec((D, D), lambda i: (0, 0))
_bias_blk = pl.BlockSpec((1, D), lambda i: (0, 0))
_acc_blk = pl.BlockSpec((NC, BLK, D), lambda i: (0, i, 0))

_tc_k1 = pl.pallas_call(
    _tc_k1_body,
    grid=(N // BLK,),
    in_specs=[_row_blk, _mat_blk, _col_blk, _col_blk],
    out_specs=[_row_blk, _col_blk],
    out_shape=[jax.ShapeDtypeStruct((N, D), jnp.float32),
               jax.ShapeDtypeStruct((N, 1), jnp.float32)],
)

_tc_k2 = pl.pallas_call(
    _tc_k2_body,
    grid=(N // BLK,),
    in_specs=[_acc_blk, _row_blk, _col_blk, _bias_blk, _mat_blk],
    out_specs=_row_blk,
    out_shape=jax.ShapeDtypeStruct((N, D), jnp.float32),
)

_tc_k3 = pl.pallas_call(
    _tc_k3_body,
    grid=(N // BLK,),
    in_specs=[_acc_blk, _row_blk, _col_blk, _bias_blk],
    out_specs=_row_blk,
    out_shape=jax.ShapeDtypeStruct((N, D), jnp.float32),
)


@jax.jit
def kernel(x, edge_index, W1, b1, W2, b2):
    src = edge_index[0].astype(jnp.int32).reshape(NW, NCHUNK, CHUNK)
    dst = edge_index[1].astype(jnp.int32).reshape(NW, NCHUNK, CHUNK)
    zeros_deg = jnp.zeros((N,), jnp.float32)
    zeros_rows = jnp.zeros((ROWS_PER_TILE, D), jnp.float32)

    degp = _sc_degree(dst, zeros_deg)
    d0 = degp[:N].reshape(N, 1)
    d1 = degp[N:].reshape(N, 1)

    g1, dis = _tc_k1(x, W1, d0, d1)
    acc1 = _sc_scatter(g1, src, dst, zeros_rows).reshape(NC, N, D)
    g2 = _tc_k2(acc1, g1, dis, b1.reshape(1, D), W2)
    acc2 = _sc_scatter(g2, src, dst, zeros_rows).reshape(NC, N, D)
    return _tc_k3(acc2, g2, dis, b2.reshape(1, D))


# speedup vs baseline: 12.3877x; 12.3877x over previous
import functools

import jax
import jax.numpy as jnp
from jax import lax
from jax.experimental import pallas as pl
from jax.experimental.pallas import tpu as pltpu
from jax.experimental.pallas import tpu_sc as plsc

N = 10000
E = 320000
D = 128
NC, NS = 2, 16
NW = NC * NS
E_PER = E // NW
CHUNK = 80
NCHUNK = E_PER // CHUNK
NP = 10240
ROWS_PER_TILE = NP // NS

_MESH = plsc.VectorSubcoreMesh(core_axis_name="c", subcore_axis_name="s")
DW = 16


@functools.partial(
    pl.kernel,
    mesh=_MESH,
    out_type=jax.ShapeDtypeStruct((NC * NP, DW), jnp.float32),
    scratch_types=[
        pltpu.VMEM((CHUNK,), jnp.int32),
        pltpu.VMEM((CHUNK, DW), jnp.float32),
        pltpu.VMEM_SHARED((NP, DW), jnp.float32),
    ],
)
def _sc_degree(dst_hbm, zeros_hbm, deg_hbm, dst_v, ones_v, deg_sh):
    c = lax.axis_index("c")
    s = lax.axis_index("s")
    wid = s * NC + c
    r0 = s * ROWS_PER_TILE
    for j in range(CHUNK):
        ones_v[j] = jnp.ones((DW,), jnp.float32)
    pltpu.sync_copy(zeros_hbm, deg_sh.at[pl.ds(r0, ROWS_PER_TILE)])
    plsc.subcore_barrier()

    def step(i, carry):
        pltpu.sync_copy(dst_hbm.at[wid, i], dst_v)
        pltpu.sync_copy(ones_v, deg_sh.at[dst_v], add=True)
        return carry

    lax.fori_loop(0, NCHUNK, step, 0)
    plsc.subcore_barrier()
    pltpu.sync_copy(deg_sh.at[pl.ds(r0, ROWS_PER_TILE)],
                    deg_hbm.at[pl.ds(c * NP + r0, ROWS_PER_TILE)])


@functools.partial(
    pl.kernel,
    mesh=_MESH,
    out_type=jax.ShapeDtypeStruct((NC * NP, D), jnp.float32),
    scratch_types=[
        pltpu.VMEM((CHUNK,), jnp.int32),
        pltpu.VMEM((CHUNK,), jnp.int32),
        pltpu.VMEM((CHUNK, D), jnp.float32),
        pltpu.VMEM_SHARED((NP, D), jnp.float32),
        pltpu.SemaphoreType.DMA,
    ],
)
def _sc_scatter(g_hbm, src_hbm, dst_hbm, zrows_hbm, acc_hbm,
                src_v, dst_v, rows_v, acc_sh, sem):
    c = lax.axis_index("c")
    s = lax.axis_index("s")
    wid = s * NC + c
    r0 = s * ROWS_PER_TILE
    pltpu.sync_copy(zrows_hbm, acc_sh.at[pl.ds(r0, ROWS_PER_TILE)])
    plsc.subcore_barrier()

    def step(i, carry):
        pltpu.sync_copy(src_hbm.at[wid, i], src_v)
        pltpu.sync_copy(dst_hbm.at[wid, i], dst_v)
        pltpu.async_copy(g_hbm.at[src_v], rows_v, sem).wait()
        pltpu.sync_copy(rows_v, acc_sh.at[dst_v], add=True)
        return carry

    lax.fori_loop(0, NCHUNK, step, 0)
    plsc.subcore_barrier()
    pltpu.sync_copy(acc_sh.at[pl.ds(r0, ROWS_PER_TILE)],
                    acc_hbm.at[pl.ds(c * NP + r0, ROWS_PER_TILE)])


BLK = 1000


def _tc_k1_body(x_ref, w_ref, d0_ref, d1_ref, g_ref, dis_ref):
    h = jnp.dot(x_ref[...], w_ref[...], preferred_element_type=jnp.float32)
    dis = lax.rsqrt(d0_ref[:, :1] + d1_ref[:, :1] + 1.0)
    g_ref[...] = h * dis
    dis_ref[...] = dis


def _tc_k2_body(acc_ref, g1_ref, dis_ref, b1_ref, w2_ref, g2_ref):
    dis = dis_ref[...]
    out1 = dis * (acc_ref[0] + acc_ref[1] + g1_ref[...]) + b1_ref[...]
    g2_ref[...] = jnp.dot(out1, w2_ref[...],
                          preferred_element_type=jnp.float32) * dis


def _tc_k3_body(acc_ref, g2_ref, dis_ref, b2_ref, out_ref):
    out_ref[...] = (dis_ref[...] * (acc_ref[0] + acc_ref[1] + g2_ref[...])
                    + b2_ref[...])


_row_blk = pl.BlockSpec((BLK, D), lambda i: (i, 0))
_col_blk = pl.BlockSpec((BLK, 1), lambda i: (i, 0))
_deg_blk = pl.BlockSpec((BLK, DW), lambda i: (i, 0))
_mat_blk = pl.BlockSpec((D, D), lambda i: (0, 0))
_bias_blk = pl.BlockSpec((1, D), lambda i: (0, 0))
_acc_blk = pl.BlockSpec((NC, BLK, D), lambda i: (0, i, 0))

_tc_k1 = pl.pallas_call(
    _tc_k1_body,
    grid=(N // BLK,),
    in_specs=[_row_blk, _mat_blk, _deg_blk, _deg_blk],
    out_specs=[_row_blk, _col_blk],
    out_shape=[jax.ShapeDtypeStruct((N, D), jnp.float32),
               jax.ShapeDtypeStruct((N, 1), jnp.float32)],
)

_tc_k2 = pl.pallas_call(
    _tc_k2_body,
    grid=(N // BLK,),
    in_specs=[_acc_blk, _row_blk, _col_blk, _bias_blk, _mat_blk],
    out_specs=_row_blk,
    out_shape=jax.ShapeDtypeStruct((N, D), jnp.float32),
)

_tc_k3 = pl.pallas_call(
    _tc_k3_body,
    grid=(N // BLK,),
    in_specs=[_acc_blk, _row_blk, _col_blk, _bias_blk],
    out_specs=_row_blk,
    out_shape=jax.ShapeDtypeStruct((N, D), jnp.float32),
)


@jax.jit
def kernel(x, edge_index, W1, b1, W2, b2):
    src = edge_index[0].astype(jnp.int32).reshape(NW, NCHUNK, CHUNK)
    dst = edge_index[1].astype(jnp.int32).reshape(NW, NCHUNK, CHUNK)
    zeros_deg = jnp.zeros((ROWS_PER_TILE, DW), jnp.float32)
    zeros_rows = jnp.zeros((ROWS_PER_TILE, D), jnp.float32)

    degp = _sc_degree(dst, zeros_deg).reshape(NC, NP, DW)
    d0 = degp[0, :N]
    d1 = degp[1, :N]

    g1, dis = _tc_k1(x, W1, d0, d1)
    acc1 = _sc_scatter(g1, src, dst, zeros_rows).reshape(NC, NP, D)
    g2 = _tc_k2(acc1, g1, dis, b1.reshape(1, D), W2)
    acc2 = _sc_scatter(g2, src, dst, zeros_rows).reshape(NC, NP, D)
    return _tc_k3(acc2, g2, dis, b2.reshape(1, D))


# speedup vs baseline: 14.1939x; 1.1458x over previous
import functools

import jax
import jax.numpy as jnp
from jax import lax
from jax.experimental import pallas as pl
from jax.experimental.pallas import tpu as pltpu
from jax.experimental.pallas import tpu_sc as plsc

N = 10000
E = 320000
D = 128
NC, NS = 2, 16
NW = NC * NS
E_PER = E // NW
CHUNK = 80
NCHUNK = E_PER // CHUNK
NP = 10240
ROWS_PER_TILE = NP // NS

_MESH = plsc.VectorSubcoreMesh(core_axis_name="c", subcore_axis_name="s")
DW = 16


@functools.partial(
    pl.kernel,
    mesh=_MESH,
    out_type=jax.ShapeDtypeStruct((NC * NP, DW), jnp.float32),
    scratch_types=[
        pltpu.VMEM((CHUNK,), jnp.int32),
        pltpu.VMEM((CHUNK, DW), jnp.float32),
        pltpu.VMEM_SHARED((NP, DW), jnp.float32),
    ],
)
def _sc_degree(dst_hbm, zeros_hbm, deg_hbm, dbuf0, ones_v, deg_sh):
    c = lax.axis_index("c")
    s = lax.axis_index("s")
    wid = s * NC + c
    r0 = s * ROWS_PER_TILE
    for j in range(CHUNK):
        ones_v[j] = jnp.ones((DW,), jnp.float32)
    pltpu.sync_copy(zeros_hbm, deg_sh.at[pl.ds(r0, ROWS_PER_TILE)])
    plsc.subcore_barrier()

    def step(i, carry):
        pltpu.sync_copy(dst_hbm.at[wid, i], dbuf0)
        pltpu.sync_copy(ones_v, deg_sh.at[dbuf0], add=True)
        return carry

    lax.fori_loop(0, NCHUNK, step, 0)
    plsc.subcore_barrier()
    pltpu.sync_copy(deg_sh.at[pl.ds(r0, ROWS_PER_TILE)],
                    deg_hbm.at[pl.ds(c * NP + r0, ROWS_PER_TILE)])


@functools.partial(
    pl.kernel,
    mesh=_MESH,
    out_type=jax.ShapeDtypeStruct((NC * NP, D), jnp.float32),
    scratch_types=[
        pltpu.VMEM((2, CHUNK), jnp.int32),
        pltpu.VMEM((2, CHUNK), jnp.int32),
        pltpu.VMEM((2, CHUNK, D), jnp.float32),
        pltpu.VMEM_SHARED((NP, D), jnp.float32),
        pltpu.SemaphoreType.DMA,
    ],
)
def _sc_scatter(g_hbm, src_hbm, dst_hbm, zrows_hbm, acc_hbm,
                sbuf, dbuf, rows, acc_sh, sem):
    c = lax.axis_index("c")
    s = lax.axis_index("s")
    wid = s * NC + c
    r0 = s * ROWS_PER_TILE
    pltpu.sync_copy(zrows_hbm, acc_sh.at[pl.ds(r0, ROWS_PER_TILE)])
    plsc.subcore_barrier()

    def prolog(i, carry):
        pltpu.sync_copy(src_hbm.at[wid, i], sbuf.at[0])
        pltpu.sync_copy(dst_hbm.at[wid, i], dbuf.at[0])
        pltpu.sync_copy(g_hbm.at[sbuf.at[0]], rows.at[0])
        pltpu.async_copy(rows.at[0], acc_sh.at[dbuf.at[0]], sem, add=True)
        return carry

    lax.fori_loop(0, 1, prolog, 0)

    def pair(p, carry):
        ia = 2 * p + 1
        pltpu.sync_copy(src_hbm.at[wid, ia], sbuf.at[1])
        pltpu.sync_copy(dst_hbm.at[wid, ia], dbuf.at[1])
        pltpu.sync_copy(g_hbm.at[sbuf.at[1]], rows.at[1])
        pltpu.make_async_copy(rows.at[0], acc_sh.at[dbuf.at[0]], sem).wait()
        pltpu.async_copy(rows.at[1], acc_sh.at[dbuf.at[1]], sem, add=True)
        pltpu.sync_copy(src_hbm.at[wid, ia + 1], sbuf.at[0])
        pltpu.sync_copy(dst_hbm.at[wid, ia + 1], dbuf.at[0])
        pltpu.sync_copy(g_hbm.at[sbuf.at[0]], rows.at[0])
        pltpu.make_async_copy(rows.at[1], acc_sh.at[dbuf.at[1]], sem).wait()
        pltpu.async_copy(rows.at[0], acc_sh.at[dbuf.at[0]], sem, add=True)
        return carry

    lax.fori_loop(0, (NCHUNK - 1) // 2, pair, 0)
    pltpu.make_async_copy(rows.at[0], acc_sh.at[dbuf.at[0]], sem).wait()
    plsc.subcore_barrier()
    pltpu.sync_copy(acc_sh.at[pl.ds(r0, ROWS_PER_TILE)],
                    acc_hbm.at[pl.ds(c * NP + r0, ROWS_PER_TILE)])


BLK = 1000


def _tc_k1_body(x_ref, w_ref, d0_ref, d1_ref, g_ref, dis_ref):
    h = jnp.dot(x_ref[...], w_ref[...], preferred_element_type=jnp.float32)
    dis = lax.rsqrt(d0_ref[:, :1] + d1_ref[:, :1] + 1.0)
    g_ref[...] = h * dis
    dis_ref[...] = dis


def _tc_k2_body(acc_ref, g1_ref, dis_ref, b1_ref, w2_ref, g2_ref):
    dis = dis_ref[...]
    out1 = dis * (acc_ref[0] + acc_ref[1] + g1_ref[...]) + b1_ref[...]
    g2_ref[...] = jnp.dot(out1, w2_ref[...],
                          preferred_element_type=jnp.float32) * dis


def _tc_k3_body(acc_ref, g2_ref, dis_ref, b2_ref, out_ref):
    out_ref[...] = (dis_ref[...] * (acc_ref[0] + acc_ref[1] + g2_ref[...])
                    + b2_ref[...])


_row_blk = pl.BlockSpec((BLK, D), lambda i: (i, 0))
_col_blk = pl.BlockSpec((BLK, 1), lambda i: (i, 0))
_deg_blk = pl.BlockSpec((BLK, DW), lambda i: (i, 0))
_mat_blk = pl.BlockSpec((D, D), lambda i: (0, 0))
_bias_blk = pl.BlockSpec((1, D), lambda i: (0, 0))
_acc_blk = pl.BlockSpec((NC, BLK, D), lambda i: (0, i, 0))

_tc_k1 = pl.pallas_call(
    _tc_k1_body,
    grid=(N // BLK,),
    in_specs=[_row_blk, _mat_blk, _deg_blk, _deg_blk],
    out_specs=[_row_blk, _col_blk],
    out_shape=[jax.ShapeDtypeStruct((N, D), jnp.float32),
               jax.ShapeDtypeStruct((N, 1), jnp.float32)],
)

_tc_k2 = pl.pallas_call(
    _tc_k2_body,
    grid=(N // BLK,),
    in_specs=[_acc_blk, _row_blk, _col_blk, _bias_blk, _mat_blk],
    out_specs=_row_blk,
    out_shape=jax.ShapeDtypeStruct((N, D), jnp.float32),
)

_tc_k3 = pl.pallas_call(
    _tc_k3_body,
    grid=(N // BLK,),
    in_specs=[_acc_blk, _row_blk, _col_blk, _bias_blk],
    out_specs=_row_blk,
    out_shape=jax.ShapeDtypeStruct((N, D), jnp.float32),
)


@jax.jit
def kernel(x, edge_index, W1, b1, W2, b2):
    src = edge_index[0].astype(jnp.int32).reshape(NW, NCHUNK, CHUNK)
    dst = edge_index[1].astype(jnp.int32).reshape(NW, NCHUNK, CHUNK)
    zeros_deg = jnp.zeros((ROWS_PER_TILE, DW), jnp.float32)
    zeros_rows = jnp.zeros((ROWS_PER_TILE, D), jnp.float32)

    degp = _sc_degree(dst, zeros_deg).reshape(NC, NP, DW)
    d0 = degp[0, :N]
    d1 = degp[1, :N]

    g1, dis = _tc_k1(x, W1, d0, d1)
    acc1 = _sc_scatter(g1, src, dst, zeros_rows).reshape(NC, NP, D)
    g2 = _tc_k2(acc1, g1, dis, b1.reshape(1, D), W2)
    acc2 = _sc_scatter(g2, src, dst, zeros_rows).reshape(NC, NP, D)
    return _tc_k3(acc2, g2, dis, b2.reshape(1, D))


# speedup vs baseline: 14.4737x; 1.0197x over previous
import functools

import jax
import jax.numpy as jnp
from jax import lax
from jax.experimental import pallas as pl
from jax.experimental.pallas import tpu as pltpu
from jax.experimental.pallas import tpu_sc as plsc

N = 10000
E = 320000
D = 128
NC, NS = 2, 16
NW = NC * NS
E_PER = E // NW
CHUNK = 80
NCHUNK = E_PER // CHUNK
NP = 10240
ROWS_PER_TILE = NP // NS

_MESH = plsc.VectorSubcoreMesh(core_axis_name="c", subcore_axis_name="s")
DW = 16


@functools.partial(
    pl.kernel,
    mesh=_MESH,
    out_type=jax.ShapeDtypeStruct((NC * NP, DW), jnp.float32),
    scratch_types=[
        pltpu.VMEM((2, CHUNK), jnp.int32),
        pltpu.VMEM((CHUNK, DW), jnp.float32),
        pltpu.VMEM_SHARED((NP, DW), jnp.float32),
        pltpu.SemaphoreType.DMA,
    ],
)
def _sc_degree(dst_hbm, zeros_hbm, deg_hbm, dbuf, ones_v, deg_sh, sem):
    c = lax.axis_index("c")
    s = lax.axis_index("s")
    wid = s * NC + c
    r0 = s * ROWS_PER_TILE
    for j in range(CHUNK):
        ones_v[j] = jnp.ones((DW,), jnp.float32)
    pltpu.sync_copy(zeros_hbm, deg_sh.at[pl.ds(r0, ROWS_PER_TILE)])
    plsc.subcore_barrier()

    def prolog(i, carry):
        pltpu.sync_copy(dst_hbm.at[wid, i], dbuf.at[0])
        pltpu.async_copy(ones_v, deg_sh.at[dbuf.at[0]], sem, add=True)
        return carry

    lax.fori_loop(0, 1, prolog, 0)

    def pair(p, carry):
        ia = 2 * p + 1
        pltpu.sync_copy(dst_hbm.at[wid, ia], dbuf.at[1])
        pltpu.make_async_copy(ones_v, deg_sh.at[dbuf.at[0]], sem).wait()
        pltpu.async_copy(ones_v, deg_sh.at[dbuf.at[1]], sem, add=True)
        pltpu.sync_copy(dst_hbm.at[wid, ia + 1], dbuf.at[0])
        pltpu.make_async_copy(ones_v, deg_sh.at[dbuf.at[1]], sem).wait()
        pltpu.async_copy(ones_v, deg_sh.at[dbuf.at[0]], sem, add=True)
        return carry

    lax.fori_loop(0, (NCHUNK - 1) // 2, pair, 0)
    pltpu.make_async_copy(ones_v, deg_sh.at[dbuf.at[0]], sem).wait()
    plsc.subcore_barrier()
    pltpu.sync_copy(deg_sh.at[pl.ds(r0, ROWS_PER_TILE)],
                    deg_hbm.at[pl.ds(c * NP + r0, ROWS_PER_TILE)])


@functools.partial(
    pl.kernel,
    mesh=_MESH,
    out_type=jax.ShapeDtypeStruct((NC * NP, D), jnp.float32),
    scratch_types=[
        pltpu.VMEM((2, CHUNK), jnp.int32),
        pltpu.VMEM((2, CHUNK), jnp.int32),
        pltpu.VMEM((2, CHUNK, D), jnp.float32),
        pltpu.VMEM_SHARED((NP, D), jnp.float32),
        pltpu.SemaphoreType.DMA,
    ],
)
def _sc_scatter(g_hbm, src_hbm, dst_hbm, zrows_hbm, acc_hbm,
                sbuf, dbuf, rows, acc_sh, sem):
    c = lax.axis_index("c")
    s = lax.axis_index("s")
    wid = s * NC + c
    r0 = s * ROWS_PER_TILE
    pltpu.sync_copy(zrows_hbm, acc_sh.at[pl.ds(r0, ROWS_PER_TILE)])
    plsc.subcore_barrier()

    def prolog(i, carry):
        pltpu.sync_copy(src_hbm.at[wid, i], sbuf.at[0])
        pltpu.sync_copy(dst_hbm.at[wid, i], dbuf.at[0])
        pltpu.sync_copy(g_hbm.at[sbuf.at[0]], rows.at[0])
        pltpu.async_copy(rows.at[0], acc_sh.at[dbuf.at[0]], sem, add=True)
        return carry

    lax.fori_loop(0, 1, prolog, 0)

    def pair(p, carry):
        ia = 2 * p + 1
        pltpu.sync_copy(src_hbm.at[wid, ia], sbuf.at[1])
        pltpu.sync_copy(dst_hbm.at[wid, ia], dbuf.at[1])
        pltpu.sync_copy(g_hbm.at[sbuf.at[1]], rows.at[1])
        pltpu.make_async_copy(rows.at[0], acc_sh.at[dbuf.at[0]], sem).wait()
        pltpu.async_copy(rows.at[1], acc_sh.at[dbuf.at[1]], sem, add=True)
        pltpu.sync_copy(src_hbm.at[wid, ia + 1], sbuf.at[0])
        pltpu.sync_copy(dst_hbm.at[wid, ia + 1], dbuf.at[0])
        pltpu.sync_copy(g_hbm.at[sbuf.at[0]], rows.at[0])
        pltpu.make_async_copy(rows.at[1], acc_sh.at[dbuf.at[1]], sem).wait()
        pltpu.async_copy(rows.at[0], acc_sh.at[dbuf.at[0]], sem, add=True)
        return carry

    lax.fori_loop(0, (NCHUNK - 1) // 2, pair, 0)
    pltpu.make_async_copy(rows.at[0], acc_sh.at[dbuf.at[0]], sem).wait()
    plsc.subcore_barrier()
    pltpu.sync_copy(acc_sh.at[pl.ds(r0, ROWS_PER_TILE)],
                    acc_hbm.at[pl.ds(c * NP + r0, ROWS_PER_TILE)])


BLK = 1000


def _tc_k1_body(x_ref, w_ref, d0_ref, d1_ref, g_ref, dis_ref):
    h = jnp.dot(x_ref[...], w_ref[...], preferred_element_type=jnp.float32)
    dis = lax.rsqrt(d0_ref[:, :1] + d1_ref[:, :1] + 1.0)
    g_ref[...] = h * dis
    dis_ref[...] = dis


def _tc_k2_body(acc_ref, g1_ref, dis_ref, b1_ref, w2_ref, g2_ref):
    dis = dis_ref[...]
    out1 = dis * (acc_ref[0] + acc_ref[1] + g1_ref[...]) + b1_ref[...]
    g2_ref[...] = jnp.dot(out1, w2_ref[...],
                          preferred_element_type=jnp.float32) * dis


def _tc_k3_body(acc_ref, g2_ref, dis_ref, b2_ref, out_ref):
    out_ref[...] = (dis_ref[...] * (acc_ref[0] + acc_ref[1] + g2_ref[...])
                    + b2_ref[...])


_row_blk = pl.BlockSpec((BLK, D), lambda i: (i, 0))
_col_blk = pl.BlockSpec((BLK, 1), lambda i: (i, 0))
_deg_blk = pl.BlockSpec((BLK, DW), lambda i: (i, 0))
_mat_blk = pl.BlockSpec((D, D), lambda i: (0, 0))
_bias_blk = pl.BlockSpec((1, D), lambda i: (0, 0))
_acc_blk = pl.BlockSpec((NC, BLK, D), lambda i: (0, i, 0))

_tc_k1 = pl.pallas_call(
    _tc_k1_body,
    grid=(N // BLK,),
    in_specs=[_row_blk, _mat_blk, _deg_blk, _deg_blk],
    out_specs=[_row_blk, _col_blk],
    out_shape=[jax.ShapeDtypeStruct((N, D), jnp.float32),
               jax.ShapeDtypeStruct((N, 1), jnp.float32)],
)

_tc_k2 = pl.pallas_call(
    _tc_k2_body,
    grid=(N // BLK,),
    in_specs=[_acc_blk, _row_blk, _col_blk, _bias_blk, _mat_blk],
    out_specs=_row_blk,
    out_shape=jax.ShapeDtypeStruct((N, D), jnp.float32),
)

_tc_k3 = pl.pallas_call(
    _tc_k3_body,
    grid=(N // BLK,),
    in_specs=[_acc_blk, _row_blk, _col_blk, _bias_blk],
    out_specs=_row_blk,
    out_shape=jax.ShapeDtypeStruct((N, D), jnp.float32),
)


@jax.jit
def kernel(x, edge_index, W1, b1, W2, b2):
    src = edge_index[0].astype(jnp.int32).reshape(NW, NCHUNK, CHUNK)
    dst = edge_index[1].astype(jnp.int32).reshape(NW, NCHUNK, CHUNK)
    zeros_deg = jnp.zeros((ROWS_PER_TILE, DW), jnp.float32)
    zeros_rows = jnp.zeros((ROWS_PER_TILE, D), jnp.float32)

    degp = _sc_degree(dst, zeros_deg).reshape(NC, NP, DW)
    d0 = degp[0, :N]
    d1 = degp[1, :N]

    g1, dis = _tc_k1(x, W1, d0, d1)
    acc1 = _sc_scatter(g1, src, dst, zeros_rows).reshape(NC, NP, D)
    g2 = _tc_k2(acc1, g1, dis, b1.reshape(1, D), W2)
    acc2 = _sc_scatter(g2, src, dst, zeros_rows).reshape(NC, NP, D)
    return _tc_k3(acc2, g2, dis, b2.reshape(1, D))


# speedup vs baseline: 17.2640x; 1.1928x over previous
import functools

import jax
import jax.numpy as jnp
from jax import lax
from jax.experimental import pallas as pl
from jax.experimental.pallas import tpu as pltpu
from jax.experimental.pallas import tpu_sc as plsc

N = 10000
E = 320000
D = 128
NC, NS = 2, 16
NW = NC * NS
E_PER = E // NW
CHUNK = 80
NCHUNK = E_PER // CHUNK
NP = 10240
ROWS_PER_TILE = NP // NS

_MESH = plsc.VectorSubcoreMesh(core_axis_name="c", subcore_axis_name="s")
DW = 16


@functools.partial(
    pl.kernel,
    mesh=_MESH,
    out_type=jax.ShapeDtypeStruct((NC * NP, DW), jnp.float32),
    scratch_types=[
        pltpu.VMEM((2, CHUNK), jnp.int32),
        pltpu.VMEM((CHUNK, DW), jnp.float32),
        pltpu.VMEM_SHARED((NP, DW), jnp.float32),
        pltpu.SemaphoreType.DMA,
    ],
)
def _sc_degree(dst_hbm, zeros_hbm, deg_hbm, dbuf, ones_v, deg_sh, sem):
    c = lax.axis_index("c")
    s = lax.axis_index("s")
    wid = s * NC + c
    r0 = s * ROWS_PER_TILE
    for j in range(CHUNK):
        ones_v[j] = jnp.ones((DW,), jnp.float32)
    pltpu.sync_copy(zeros_hbm, deg_sh.at[pl.ds(r0, ROWS_PER_TILE)])
    plsc.subcore_barrier()

    def prolog(i, carry):
        pltpu.sync_copy(dst_hbm.at[wid, i], dbuf.at[0])
        pltpu.async_copy(ones_v, deg_sh.at[dbuf.at[0]], sem, add=True)
        return carry

    lax.fori_loop(0, 1, prolog, 0)

    def pair(p, carry):
        ia = 2 * p + 1
        pltpu.sync_copy(dst_hbm.at[wid, ia], dbuf.at[1])
        pltpu.make_async_copy(ones_v, deg_sh.at[dbuf.at[0]], sem).wait()
        pltpu.async_copy(ones_v, deg_sh.at[dbuf.at[1]], sem, add=True)
        pltpu.sync_copy(dst_hbm.at[wid, ia + 1], dbuf.at[0])
        pltpu.make_async_copy(ones_v, deg_sh.at[dbuf.at[1]], sem).wait()
        pltpu.async_copy(ones_v, deg_sh.at[dbuf.at[0]], sem, add=True)
        return carry

    lax.fori_loop(0, (NCHUNK - 1) // 2, pair, 0)
    pltpu.make_async_copy(ones_v, deg_sh.at[dbuf.at[0]], sem).wait()
    plsc.subcore_barrier()
    pltpu.sync_copy(deg_sh.at[pl.ds(r0, ROWS_PER_TILE)],
                    deg_hbm.at[pl.ds(c * NP + r0, ROWS_PER_TILE)])


@functools.partial(
    pl.kernel,
    mesh=_MESH,
    out_type=jax.ShapeDtypeStruct((NC * NP, D), jnp.float32),
    scratch_types=[
        pltpu.VMEM((NCHUNK, CHUNK), jnp.int32),
        pltpu.VMEM((2, CHUNK), jnp.int32),
        pltpu.VMEM((2, CHUNK, D), jnp.float32),
        pltpu.VMEM_SHARED((NP, D), jnp.float32),
        pltpu.SemaphoreType.DMA,
    ],
)
def _sc_scatter(g_hbm, src_hbm, dst_hbm, zrows_hbm, acc_hbm,
                slab, dbuf, rows, acc_sh, sem):
    c = lax.axis_index("c")
    s = lax.axis_index("s")
    wid = s * NC + c
    r0 = s * ROWS_PER_TILE
    pltpu.sync_copy(zrows_hbm, acc_sh.at[pl.ds(r0, ROWS_PER_TILE)])
    plsc.subcore_barrier()

    pltpu.sync_copy(src_hbm.at[wid], slab)

    def prolog(i, carry):
        pltpu.sync_copy(dst_hbm.at[wid, i], dbuf.at[0])
        pltpu.sync_copy(g_hbm.at[slab.at[i]], rows.at[0])
        pltpu.async_copy(rows.at[0], acc_sh.at[dbuf.at[0]], sem, add=True)
        return carry

    lax.fori_loop(0, 1, prolog, 0)

    def pair(p, carry):
        ia = 2 * p + 1
        pltpu.sync_copy(dst_hbm.at[wid, ia], dbuf.at[1])
        pltpu.sync_copy(g_hbm.at[slab.at[ia]], rows.at[1])
        pltpu.make_async_copy(rows.at[0], acc_sh.at[dbuf.at[0]], sem).wait()
        pltpu.async_copy(rows.at[1], acc_sh.at[dbuf.at[1]], sem, add=True)
        pltpu.sync_copy(dst_hbm.at[wid, ia + 1], dbuf.at[0])
        pltpu.sync_copy(g_hbm.at[slab.at[ia + 1]], rows.at[0])
        pltpu.make_async_copy(rows.at[1], acc_sh.at[dbuf.at[1]], sem).wait()
        pltpu.async_copy(rows.at[0], acc_sh.at[dbuf.at[0]], sem, add=True)
        return carry

    lax.fori_loop(0, (NCHUNK - 1) // 2, pair, 0)
    pltpu.make_async_copy(rows.at[0], acc_sh.at[dbuf.at[0]], sem).wait()
    plsc.subcore_barrier()
    pltpu.sync_copy(acc_sh.at[pl.ds(r0, ROWS_PER_TILE)],
                    acc_hbm.at[pl.ds(c * NP + r0, ROWS_PER_TILE)])


BLK = 1000


def _tc_k1_body(x_ref, w_ref, d0_ref, d1_ref, g_ref, dis_ref):
    h = jnp.dot(x_ref[...], w_ref[...], preferred_element_type=jnp.float32)
    dis = lax.rsqrt(d0_ref[:, :1] + d1_ref[:, :1] + 1.0)
    g_ref[...] = h * dis
    dis_ref[...] = dis


def _tc_k2_body(acc_ref, g1_ref, dis_ref, b1_ref, w2_ref, g2_ref):
    dis = dis_ref[...]
    out1 = dis * (acc_ref[0] + acc_ref[1] + g1_ref[...]) + b1_ref[...]
    g2_ref[...] = jnp.dot(out1, w2_ref[...],
                          preferred_element_type=jnp.float32) * dis


def _tc_k3_body(acc_ref, g2_ref, dis_ref, b2_ref, out_ref):
    out_ref[...] = (dis_ref[...] * (acc_ref[0] + acc_ref[1] + g2_ref[...])
                    + b2_ref[...])


_row_blk = pl.BlockSpec((BLK, D), lambda i: (i, 0))
_col_blk = pl.BlockSpec((BLK, 1), lambda i: (i, 0))
_deg_blk = pl.BlockSpec((BLK, DW), lambda i: (i, 0))
_mat_blk = pl.BlockSpec((D, D), lambda i: (0, 0))
_bias_blk = pl.BlockSpec((1, D), lambda i: (0, 0))
_acc_blk = pl.BlockSpec((NC, BLK, D), lambda i: (0, i, 0))

_tc_k1 = pl.pallas_call(
    _tc_k1_body,
    grid=(N // BLK,),
    in_specs=[_row_blk, _mat_blk, _deg_blk, _deg_blk],
    out_specs=[_row_blk, _col_blk],
    out_shape=[jax.ShapeDtypeStruct((N, D), jnp.float32),
               jax.ShapeDtypeStruct((N, 1), jnp.float32)],
)

_tc_k2 = pl.pallas_call(
    _tc_k2_body,
    grid=(N // BLK,),
    in_specs=[_acc_blk, _row_blk, _col_blk, _bias_blk, _mat_blk],
    out_specs=_row_blk,
    out_shape=jax.ShapeDtypeStruct((N, D), jnp.float32),
)

_tc_k3 = pl.pallas_call(
    _tc_k3_body,
    grid=(N // BLK,),
    in_specs=[_acc_blk, _row_blk, _col_blk, _bias_blk],
    out_specs=_row_blk,
    out_shape=jax.ShapeDtypeStruct((N, D), jnp.float32),
)


@jax.jit
def kernel(x, edge_index, W1, b1, W2, b2):
    src = edge_index[0].astype(jnp.int32).reshape(NW, NCHUNK, CHUNK)
    dst = edge_index[1].astype(jnp.int32).reshape(NW, NCHUNK, CHUNK)
    zeros_deg = jnp.zeros((ROWS_PER_TILE, DW), jnp.float32)
    zeros_rows = jnp.zeros((ROWS_PER_TILE, D), jnp.float32)

    degp = _sc_degree(dst, zeros_deg).reshape(NC, NP, DW)
    d0 = degp[0, :N]
    d1 = degp[1, :N]

    g1, dis = _tc_k1(x, W1, d0, d1)
    acc1 = _sc_scatter(g1, src, dst, zeros_rows).reshape(NC, NP, D)
    g2 = _tc_k2(acc1, g1, dis, b1.reshape(1, D), W2)
    acc2 = _sc_scatter(g2, src, dst, zeros_rows).reshape(NC, NP, D)
    return _tc_k3(acc2, g2, dis, b2.reshape(1, D))


# speedup vs baseline: 17.3967x; 1.0077x over previous
import functools

import jax
import jax.numpy as jnp
from jax import lax
from jax.experimental import pallas as pl
from jax.experimental.pallas import tpu as pltpu
from jax.experimental.pallas import tpu_sc as plsc

N = 10000
E = 320000
D = 128
NC, NS = 2, 16
NW = NC * NS
E_PER = E // NW
CHUNK = 80
NCHUNK = E_PER // CHUNK
NP = 10240
ROWS_PER_TILE = NP // NS

_MESH = plsc.VectorSubcoreMesh(core_axis_name="c", subcore_axis_name="s")
DW = 16


@functools.partial(
    pl.kernel,
    mesh=_MESH,
    out_type=jax.ShapeDtypeStruct((NC * NP, DW), jnp.float32),
    scratch_types=[
        pltpu.VMEM((2, CHUNK), jnp.int32),
        pltpu.VMEM((CHUNK, DW), jnp.float32),
        pltpu.VMEM_SHARED((NP, DW), jnp.float32),
        pltpu.SemaphoreType.DMA,
    ],
)
def _sc_degree(dst_hbm, zeros_hbm, deg_hbm, dbuf, ones_v, deg_sh, sem):
    c = lax.axis_index("c")
    s = lax.axis_index("s")
    wid = s * NC + c
    r0 = s * ROWS_PER_TILE
    for j in range(CHUNK):
        ones_v[j] = jnp.ones((DW,), jnp.float32)
    pltpu.sync_copy(zeros_hbm, deg_sh.at[pl.ds(r0, ROWS_PER_TILE)])
    plsc.subcore_barrier()

    def prolog(i, carry):
        pltpu.sync_copy(dst_hbm.at[wid, i], dbuf.at[0])
        pltpu.async_copy(ones_v, deg_sh.at[dbuf.at[0]], sem, add=True)
        return carry

    lax.fori_loop(0, 1, prolog, 0)

    def pair(p, carry):
        ia = 2 * p + 1
        pltpu.sync_copy(dst_hbm.at[wid, ia], dbuf.at[1])
        pltpu.make_async_copy(ones_v, deg_sh.at[dbuf.at[0]], sem).wait()
        pltpu.async_copy(ones_v, deg_sh.at[dbuf.at[1]], sem, add=True)
        pltpu.sync_copy(dst_hbm.at[wid, ia + 1], dbuf.at[0])
        pltpu.make_async_copy(ones_v, deg_sh.at[dbuf.at[1]], sem).wait()
        pltpu.async_copy(ones_v, deg_sh.at[dbuf.at[0]], sem, add=True)
        return carry

    lax.fori_loop(0, (NCHUNK - 1) // 2, pair, 0)
    pltpu.make_async_copy(ones_v, deg_sh.at[dbuf.at[0]], sem).wait()
    plsc.subcore_barrier()
    pltpu.sync_copy(deg_sh.at[pl.ds(r0, ROWS_PER_TILE)],
                    deg_hbm.at[pl.ds(c * NP + r0, ROWS_PER_TILE)])


@functools.partial(
    pl.kernel,
    mesh=_MESH,
    out_type=jax.ShapeDtypeStruct((NC * NP, D), jnp.float32),
    scratch_types=[
        pltpu.VMEM((NCHUNK, CHUNK), jnp.int32),
        pltpu.VMEM((2, CHUNK), jnp.int32),
        pltpu.VMEM((2, CHUNK, D), jnp.float32),
        pltpu.VMEM_SHARED((NP, D), jnp.float32),
        pltpu.SemaphoreType.DMA,
    ],
)
def _sc_scatter(g_hbm, src_hbm, dst_hbm, zrows_hbm, acc_hbm,
                slab, dbuf, rows, acc_sh, sem):
    c = lax.axis_index("c")
    s = lax.axis_index("s")
    wid = s * NC + c
    r0 = s * ROWS_PER_TILE
    pltpu.sync_copy(zrows_hbm, acc_sh.at[pl.ds(r0, ROWS_PER_TILE)])
    plsc.subcore_barrier()

    pltpu.sync_copy(src_hbm.at[wid], slab)

    def prolog(i, carry):
        pltpu.sync_copy(dst_hbm.at[wid, i], dbuf.at[0])
        pltpu.sync_copy(g_hbm.at[slab.at[i]], rows.at[0])
        pltpu.async_copy(rows.at[0], acc_sh.at[dbuf.at[0]], sem, add=True)
        return carry

    lax.fori_loop(0, 1, prolog, 0)

    def pair(p, carry):
        ia = 2 * p + 1
        pltpu.sync_copy(dst_hbm.at[wid, ia], dbuf.at[1])
        pltpu.sync_copy(g_hbm.at[slab.at[ia]], rows.at[1])
        pltpu.make_async_copy(rows.at[0], acc_sh.at[dbuf.at[0]], sem).wait()
        pltpu.async_copy(rows.at[1], acc_sh.at[dbuf.at[1]], sem, add=True)
        pltpu.sync_copy(dst_hbm.at[wid, ia + 1], dbuf.at[0])
        pltpu.sync_copy(g_hbm.at[slab.at[ia + 1]], rows.at[0])
        pltpu.make_async_copy(rows.at[1], acc_sh.at[dbuf.at[1]], sem).wait()
        pltpu.async_copy(rows.at[0], acc_sh.at[dbuf.at[0]], sem, add=True)
        return carry

    lax.fori_loop(0, (NCHUNK - 1) // 2, pair, 0)
    pltpu.make_async_copy(rows.at[0], acc_sh.at[dbuf.at[0]], sem).wait()
    plsc.subcore_barrier()
    pltpu.sync_copy(acc_sh.at[pl.ds(r0, ROWS_PER_TILE)],
                    acc_hbm.at[pl.ds(c * NP + r0, ROWS_PER_TILE)])


BLK = 1000


def _tc_k0_body(x_ref, w_ref, h_ref):
    h_ref[...] = jnp.dot(x_ref[...], w_ref[...],
                         preferred_element_type=jnp.float32)


def _tc_k1_body(h_ref, degp_ref, g_ref, dis_ref):
    dis = lax.rsqrt(degp_ref[0, :, :1] + degp_ref[1, :, :1] + 1.0)
    g_ref[...] = h_ref[...] * dis
    dis_ref[...] = dis


def _tc_k2_body(acc_ref, g1_ref, dis_ref, b1_ref, w2_ref, g2_ref):
    dis = dis_ref[...]
    out1 = dis * (acc_ref[0] + acc_ref[1] + g1_ref[...]) + b1_ref[...]
    g2_ref[...] = jnp.dot(out1, w2_ref[...],
                          preferred_element_type=jnp.float32) * dis


def _tc_k3_body(acc_ref, g2_ref, dis_ref, b2_ref, out_ref):
    out_ref[...] = (dis_ref[...] * (acc_ref[0] + acc_ref[1] + g2_ref[...])
                    + b2_ref[...])


_row_blk = pl.BlockSpec((BLK, D), lambda i: (i, 0))
_col_blk = pl.BlockSpec((BLK, 1), lambda i: (i, 0))
_deg_blk = pl.BlockSpec((BLK, DW), lambda i: (i, 0))
_mat_blk = pl.BlockSpec((D, D), lambda i: (0, 0))
_bias_blk = pl.BlockSpec((1, D), lambda i: (0, 0))
_acc_blk = pl.BlockSpec((NC, BLK, D), lambda i: (0, i, 0))
_degp_blk = pl.BlockSpec((NC, BLK, DW), lambda i: (0, i, 0))

_tc_k0 = pl.pallas_call(
    _tc_k0_body,
    grid=(N // BLK,),
    in_specs=[_row_blk, _mat_blk],
    out_specs=_row_blk,
    out_shape=jax.ShapeDtypeStruct((N, D), jnp.float32),
)

_tc_k1 = pl.pallas_call(
    _tc_k1_body,
    grid=(N // BLK,),
    in_specs=[_row_blk, _degp_blk],
    out_specs=[_row_blk, _col_blk],
    out_shape=[jax.ShapeDtypeStruct((N, D), jnp.float32),
               jax.ShapeDtypeStruct((N, 1), jnp.float32)],
)

_tc_k2 = pl.pallas_call(
    _tc_k2_body,
    grid=(N // BLK,),
    in_specs=[_acc_blk, _row_blk, _col_blk, _bias_blk, _mat_blk],
    out_specs=_row_blk,
    out_shape=jax.ShapeDtypeStruct((N, D), jnp.float32),
)

_tc_k3 = pl.pallas_call(
    _tc_k3_body,
    grid=(N // BLK,),
    in_specs=[_acc_blk, _row_blk, _col_blk, _bias_blk],
    out_specs=_row_blk,
    out_shape=jax.ShapeDtypeStruct((N, D), jnp.float32),
)


@jax.jit
def kernel(x, edge_index, W1, b1, W2, b2):
    src = edge_index[0].astype(jnp.int32).reshape(NW, NCHUNK, CHUNK)
    dst = edge_index[1].astype(jnp.int32).reshape(NW, NCHUNK, CHUNK)
    zeros_deg = jnp.zeros((ROWS_PER_TILE, DW), jnp.float32)
    zeros_rows = jnp.zeros((ROWS_PER_TILE, D), jnp.float32)

    h1 = _tc_k0(x, W1)
    degp = _sc_degree(dst, zeros_deg).reshape(NC, NP, DW)
    g1, dis = _tc_k1(h1, degp)
    acc1 = _sc_scatter(g1, src, dst, zeros_rows).reshape(NC, NP, D)
    g2 = _tc_k2(acc1, g1, dis, b1.reshape(1, D), W2)
    acc2 = _sc_scatter(g2, src, dst, zeros_rows).reshape(NC, NP, D)
    return _tc_k3(acc2, g2, dis, b2.reshape(1, D))
